# Initial kernel scaffold; baseline (speedup 1.0000x reference)
import jax
import jax.numpy as jnp
from jax.experimental import pallas as pl


def kernel(output1, output2, quant):
    raise NotImplementedError("write your pallas kernel here")


# speedup vs baseline: 6.0241x; 6.0241x over previous
import functools

import jax
import jax.numpy as jnp
from jax import lax
from jax.experimental import pallas as pl
from jax.experimental.pallas import tpu as pltpu

MARGIN_ = 2.0
BLOCK_I = 256
BLOCK_J = 1024
N_ITERS = 30


def _loss_body(o1_ref, o2t_ref, rn_ref, rna_ref, out_ref, key_sc):
    b = pl.program_id(0)
    j = pl.program_id(1)
    nj = pl.num_programs(1)

    @pl.when((b == 0) & (j == 0))
    def _init():
        out_ref[...] = jnp.zeros_like(out_ref)

    o1 = o1_ref[...]
    o2t = o2t_ref[...]
    dot = lax.dot_general(o1, o2t, (((1,), (0,)), ((), ())),
                          preferred_element_type=jnp.float32)
    o2sq = jnp.sum(o2t * o2t, axis=0)
    key_sc[:, pl.ds(j * BLOCK_J, BLOCK_J)] = o2sq[None, :] - 2.0 * dot

    @pl.when(j == nj - 1)
    def _select():
        n = key_sc.shape[1]
        n_ch = n // BLOCK_J
        o1sq = jnp.sum(o1 * o1, axis=1, keepdims=True)
        grow = b * BLOCK_I + lax.broadcasted_iota(
            jnp.int32, (BLOCK_I, 1), 0)

        def chunk(cc):
            return key_sc[:, pl.ds(cc * BLOCK_J, BLOCK_J)]

        def stats_body(cc, carry):
            mn, mx, dk = carry
            ch = chunk(cc)
            cols = cc * BLOCK_J + lax.broadcasted_iota(
                jnp.int32, (BLOCK_I, BLOCK_J), 1)
            dk += jnp.sum(jnp.where(cols == grow, ch, 0.0), axis=1,
                          keepdims=True)
            mn = jnp.minimum(mn, jnp.min(ch, axis=1, keepdims=True))
            mx = jnp.maximum(mx, jnp.max(ch, axis=1, keepdims=True))
            return mn, mx, dk

        mn, mx, dkey = lax.fori_loop(
            0, n_ch, stats_body,
            (jnp.full((BLOCK_I, 1), jnp.inf, jnp.float32),
             jnp.full((BLOCK_I, 1), -jnp.inf, jnp.float32),
             jnp.zeros((BLOCK_I, 1), jnp.float32)))
        pos = o1sq + dkey

        def rank_body(cc, r):
            return r + jnp.sum((chunk(cc) < dkey).astype(jnp.int32),
                               axis=1, keepdims=True)

        r = lax.fori_loop(0, n_ch, rank_body,
                          jnp.zeros((BLOCK_I, 1), jnp.int32))
        rn = rn_ref[0, 0, :].reshape(BLOCK_I, 1)
        rna = rna_ref[0, 0, :].reshape(BLOCK_I, 1)
        t = jnp.where(r == rn, rna, rn)

        def bisect_body(_, carry):
            lo, hi = carry
            mid = 0.5 * (lo + hi)

            def count_body(cc, c):
                return c + jnp.sum((chunk(cc) <= mid).astype(jnp.int32),
                                   axis=1, keepdims=True)

            c = lax.fori_loop(0, n_ch, count_body,
                              jnp.zeros((BLOCK_I, 1), jnp.int32))
            pred = c >= t + 1
            return jnp.where(pred, lo, mid), jnp.where(pred, mid, hi)

        lo, _ = lax.fori_loop(0, N_ITERS, bisect_body, (mn - 1.0, mx))

        def vmin_body(cc, v):
            ch = chunk(cc)
            return jnp.minimum(v, jnp.min(
                jnp.where(ch > lo, ch, jnp.inf), axis=1, keepdims=True))

        v = lax.fori_loop(0, n_ch, vmin_body,
                          jnp.full((BLOCK_I, 1), jnp.inf, jnp.float32))
        neg_d = jnp.sqrt(jnp.maximum(o1sq + v, 1e-12))
        neg = jnp.maximum(MARGIN_ - neg_d, 0.0)
        bsum = (jnp.sum(pos) + jnp.sum(neg)) / jnp.float32(n)
        out_ref[...] += bsum


@functools.partial(jax.jit, static_argnames=())
def kernel(output1, output2, quant):
    n, k = output1.shape
    q = jnp.minimum(quant, n - 1)
    rkey = jax.random.key(42)
    rn = jax.random.randint(rkey, (n,), 0, q)
    rna = (rn + 1) % q
    nb_i = n // BLOCK_I
    nb_j = n // BLOCK_J
    rn3 = rn.astype(jnp.int32).reshape(nb_i, 1, BLOCK_I)
    rna3 = rna.astype(jnp.int32).reshape(nb_i, 1, BLOCK_I)

    out = pl.pallas_call(
        _loss_body,
        grid=(nb_i, nb_j),
        in_specs=[
            pl.BlockSpec((BLOCK_I, k), lambda i, j: (i, 0)),
            pl.BlockSpec((k, BLOCK_J), lambda i, j: (0, j)),
            pl.BlockSpec((1, 1, BLOCK_I), lambda i, j: (i, 0, 0)),
            pl.BlockSpec((1, 1, BLOCK_I), lambda i, j: (i, 0, 0)),
        ],
        out_specs=pl.BlockSpec((8, 128), lambda i, j: (0, 0)),
        out_shape=jax.ShapeDtypeStruct((8, 128), jnp.float32),
        scratch_shapes=[pltpu.VMEM((BLOCK_I, n), jnp.float32)],
    )(output1, output2.T, rn3, rna3)
    return out[0, 0]
